# Initial kernel scaffold; baseline (speedup 1.0000x reference)
import jax
import jax.numpy as jnp
from jax.experimental import pallas as pl


def kernel(logits_seq, feats, rel_ids, yes_idx, no_idx, mu, cnt):
    raise NotImplementedError("write your pallas kernel here")


# speedup vs baseline: 2.3535x; 2.3535x over previous
import functools

import jax
import jax.numpy as jnp
from jax import lax
from jax.experimental import pallas as pl
from jax.experimental.pallas import tpu as pltpu
from jax.experimental.pallas import tpu_sc as plsc

WEIGHT = 0.2
WARMUP = 50


def _make_sc_gather(B, R, D):
    info = plsc.get_sparse_core_info()
    NC, NS = info.num_cores, info.num_subcores
    NW = NC * NS
    assert B % NW == 0
    b_per_w = B // NW
    CH = 128
    assert b_per_w % CH == 0
    n_chunks = b_per_w // CH
    NB = 2

    mesh = plsc.VectorSubcoreMesh(core_axis_name="c", subcore_axis_name="s")

    @functools.partial(
        pl.kernel,
        mesh=mesh,
        compiler_params=pltpu.CompilerParams(needs_layout_passes=False),
        out_type=(
            jax.ShapeDtypeStruct((B, D), jnp.float32),
            jax.ShapeDtypeStruct((B,), jnp.int32),
        ),
        scratch_types=[
            pltpu.VMEM((b_per_w,), jnp.int32),
            pltpu.VMEM((NB, CH, D), jnp.float32),
            pltpu.VMEM((R,), jnp.int32),
            pltpu.VMEM((b_per_w,), jnp.int32),
            pltpu.SemaphoreType.DMA,
            pltpu.SemaphoreType.DMA,
            pltpu.SemaphoreType.DMA,
            pltpu.SemaphoreType.DMA,
        ],
    )
    def sc_gather(ids_hbm, mu_hbm, cnt_hbm, musel_hbm, cntsel_hbm,
                  idx_v, mu_v, cnt_v, cntsel_v, gsem0, gsem1, ssem0, ssem1):
        wid = lax.axis_index("s") * NC + lax.axis_index("c")
        base = wid * b_per_w
        pltpu.sync_copy(ids_hbm.at[pl.ds(base, b_per_w)], idx_v)
        pltpu.sync_copy(cnt_hbm, cnt_v)
        gsems = (gsem0, gsem1)
        ssems = (ssem0, ssem1)

        def gather(c, slot):
            idx = idx_v.at[pl.ds(c * CH, CH)]
            pltpu.async_copy(mu_hbm.at[idx], mu_v.at[slot], gsems[slot])

        def drain_gather(slot):
            pltpu.make_async_copy(mu_hbm.at[idx_v.at[pl.ds(0, CH)]],
                                  mu_v.at[slot], gsems[slot]).wait()

        def put(c, slot):
            row0 = base + c * CH
            pltpu.async_copy(mu_v.at[slot], musel_hbm.at[pl.ds(row0, CH)],
                             ssems[slot])

        def drain_put(slot):
            pltpu.make_async_copy(mu_v.at[slot],
                                  musel_hbm.at[pl.ds(0, CH)],
                                  ssems[slot]).wait()

        gather(0, 0)
        if n_chunks > 1:
            gather(1, 1)
        for c in range(n_chunks):
            slot = c % NB
            drain_gather(slot)
            put(c, slot)
            if c + NB < n_chunks:
                drain_put(slot)
                gather(c + NB, slot)
        L = 16
        for g in range(b_per_w // L):
            ids16 = idx_v[pl.ds(g * L, L)]
            cntsel_v[pl.ds(g * L, L)] = plsc.load_gather(cnt_v, [ids16])
        pltpu.sync_copy(cntsel_v, cntsel_hbm.at[pl.ds(base, b_per_w)])
        for c in range(max(0, n_chunks - NB), n_chunks):
            drain_put(c % NB)

    return sc_gather


def _apply_body(logits_ref, feats_ref, musel_ref, cntsel_ref, mask_ref,
                out_ref):
    f = feats_ref[...]
    m = musel_ref[...]
    dot = jnp.sum(f * m, axis=1, keepdims=True)
    ff = jnp.sum(f * f, axis=1, keepdims=True)
    mm = jnp.sum(m * m, axis=1, keepdims=True)
    inv = lax.rsqrt(jnp.maximum(ff, 1e-16) * jnp.maximum(mm, 1e-16))
    gate = cntsel_ref[...] >= WARMUP
    bias = jnp.where(gate, WEIGHT * dot * inv, 0.0)
    out_ref[...] = logits_ref[...] + bias * mask_ref[...]


def kernel(logits_seq, feats, rel_ids, yes_idx, no_idx, mu, cnt):
    B, V = logits_seq.shape
    R, D = mu.shape
    r = jnp.clip(rel_ids, 0, R - 1).astype(jnp.int32)

    mu_sel, cnt_sel = _make_sc_gather(B, R, D)(r, mu, cnt)
    cnt_sel = cnt_sel.reshape(B, 1)

    cols = lax.iota(jnp.int32, V)[None, :]
    mask_row = ((cols == yes_idx).astype(jnp.float32)
                - (cols == no_idx).astype(jnp.float32))

    BR = 512
    out = pl.pallas_call(
        _apply_body,
        grid=(B // BR,),
        in_specs=[
            pl.BlockSpec((BR, V), lambda i: (i, 0)),
            pl.BlockSpec((BR, D), lambda i: (i, 0)),
            pl.BlockSpec((BR, D), lambda i: (i, 0)),
            pl.BlockSpec((BR, 1), lambda i: (i, 0)),
            pl.BlockSpec((1, V), lambda i: (0, 0)),
        ],
        out_specs=pl.BlockSpec((BR, V), lambda i: (i, 0)),
        out_shape=jax.ShapeDtypeStruct((B, V), jnp.float32),
    )(logits_seq, feats, mu_sel, cnt_sel, mask_row)
    return out
